# baseline (speedup 1.0000x reference)
import functools

import jax
import jax.numpy as jnp
from jax import lax
from jax.experimental import pallas as pl
from jax.experimental.pallas import tpu as pltpu
from jax.experimental.pallas import tpu_sc as plsc

_NEW_TOKEN_ID = 1000000
_VOCAB = 1000000
_D = 64

_NC = 2
_NS = 16
_NW = _NC * _NS

_C = 512
_SLAB = 128
_NSLAB = _C // _SLAB
_G = _C // 16


@functools.partial(jax.jit, static_argnums=(3,))
def _lookup(ids, table, new_row, batch):
    per_w = batch // _NW
    n_chunks = per_w // _C
    mesh = plsc.VectorSubcoreMesh(core_axis_name="c", subcore_axis_name="s")

    @functools.partial(
        pl.kernel,
        mesh=mesh,
        out_type=jax.ShapeDtypeStruct((batch, _D), jnp.float32),
        scratch_types=[
            pltpu.VMEM((_C,), jnp.int32),
            pltpu.VMEM((_NSLAB, _SLAB), jnp.int32),
            pltpu.VMEM((_C, _D), jnp.float32),
            pltpu.VMEM((_D,), jnp.float32),
            pltpu.SemaphoreType.DMA,
        ],
        compiler_params=pltpu.CompilerParams(
            needs_layout_passes=False, use_tc_tiling_on_sc=False
        ),
    )
    def k(ids_hbm, table_hbm, new_hbm, out_hbm, ids_v, safe_v, rows_v, new_v, sem):
        wid = lax.axis_index("s") * _NC + lax.axis_index("c")
        wbase = wid * per_w
        pltpu.sync_copy(new_hbm, new_v)

        def chunk(ci, carry):
            cbase = wbase + ci * _C
            pltpu.sync_copy(ids_hbm.at[pl.ds(cbase, _C)], ids_v)

            mx = None
            for g in range(_G):
                idv = ids_v[pl.ds(g * 16, 16)]
                j, c = divmod(g * 16, _SLAB)
                safe_v[j, pl.ds(c, 16)] = jnp.minimum(idv, _VOCAB - 1)
                mx = idv if mx is None else jnp.maximum(mx, idv)

            cps = [
                pltpu.async_copy(
                    table_hbm.at[safe_v.at[j]],
                    rows_v.at[pl.ds(j * _SLAB, _SLAB)],
                    sem,
                )
                for j in range(_NSLAB)
            ]
            for cp in cps:
                cp.wait()

            @pl.when(jnp.max(mx) >= _NEW_TOKEN_ID)
            def _fixup():
                liota = lax.iota(jnp.int32, 16)
                for g in range(_G):
                    idv = ids_v[pl.ds(g * 16, 16)]
                    m = idv == _NEW_TOKEN_ID
                    rowv = g * 16 + liota
                    for q in range(_D // 16):
                        plsc.store_scatter(
                            rows_v,
                            [rowv, q * 16 + liota],
                            new_v[pl.ds(q * 16, 16)],
                            mask=m,
                        )

            pltpu.sync_copy(rows_v, out_hbm.at[pl.ds(cbase, _C)])
            return carry

        lax.fori_loop(0, n_chunks, chunk, 0)

    return k(ids, table, new_row)


def kernel(input_ids, base_weight, new_weight):
    b, s = input_ids.shape
    ids = input_ids.reshape(b * s).astype(jnp.int32)
    out = _lookup(ids, base_weight, new_weight.reshape(_D), b * s)
    return out.reshape(b, s, _D)


# speedup vs baseline: 1.0430x; 1.0430x over previous
import functools

import jax
import jax.numpy as jnp
from jax import lax
from jax.experimental import pallas as pl
from jax.experimental.pallas import tpu as pltpu
from jax.experimental.pallas import tpu_sc as plsc

_NEW_TOKEN_ID = 1000000
_VOCAB = 1000000
_D = 64

_NC = 2
_NS = 16
_NW = _NC * _NS

_C = 512
_SLAB = 128
_NSLAB = _C // _SLAB
_G = _C // 16


@functools.partial(jax.jit, static_argnums=(3,))
def _lookup(ids, table, new_row, batch):
    per_w = batch // _NW
    n_chunks = per_w // _C
    n_pairs = n_chunks // 2
    mesh = plsc.VectorSubcoreMesh(core_axis_name="c", subcore_axis_name="s")

    @functools.partial(
        pl.kernel,
        mesh=mesh,
        out_type=jax.ShapeDtypeStruct((batch, _D), jnp.float32),
        scratch_types=[
            pltpu.VMEM((per_w,), jnp.int32),
            pltpu.VMEM((_NSLAB, _SLAB), jnp.int32),
            pltpu.VMEM((_NSLAB, _SLAB), jnp.int32),
            pltpu.VMEM((_C, _D), jnp.float32),
            pltpu.VMEM((_C, _D), jnp.float32),
            pltpu.VMEM((_D,), jnp.float32),
            pltpu.SMEM((2,), jnp.int32),
            pltpu.SemaphoreType.DMA,
            pltpu.SemaphoreType.DMA,
            pltpu.SemaphoreType.DMA,
            pltpu.SemaphoreType.DMA,
        ],
        compiler_params=pltpu.CompilerParams(
            needs_layout_passes=False, use_tc_tiling_on_sc=False
        ),
    )
    def k(ids_hbm, table_hbm, new_hbm, out_hbm, ids_v, safe0, safe1,
          rows0, rows1, new_v, flags, gsem0, gsem1, osem0, osem1):
        wid = lax.axis_index("s") * _NC + lax.axis_index("c")
        wbase = wid * per_w
        safe = (safe0, safe1)
        rows = (rows0, rows1)
        gsem = (gsem0, gsem1)
        osem = (osem0, osem1)

        pltpu.sync_copy(new_hbm, new_v)
        pltpu.sync_copy(ids_hbm.at[pl.ds(wbase, per_w)], ids_v)

        def pass1(ci, b):
            base = ci * _C
            mx = None
            for g in range(_G):
                idv = ids_v[pl.ds(base + g * 16, 16)]
                j, c = divmod(g * 16, _SLAB)
                safe[b][j, pl.ds(c, 16)] = jnp.minimum(idv, _VOCAB - 1)
                mx = idv if mx is None else jnp.maximum(mx, idv)
            flags[b] = jnp.max(mx)

        def gather_cps(b):
            return [
                pltpu.make_async_copy(
                    table_hbm.at[safe[b].at[j]],
                    rows[b].at[pl.ds(j * _SLAB, _SLAB)],
                    gsem[b],
                )
                for j in range(_NSLAB)
            ]

        def fire_gathers(b):
            for j in range(_NSLAB):
                pltpu.async_copy(
                    table_hbm.at[safe[b].at[j]],
                    rows[b].at[pl.ds(j * _SLAB, _SLAB)],
                    gsem[b],
                )

        def drain_gathers(b):
            for cp in gather_cps(b):
                cp.wait()

        def out_cp(ci, b):
            return pltpu.make_async_copy(
                rows[b], out_hbm.at[pl.ds(wbase + ci * _C, _C)], osem[b]
            )

        def fixup(ci, b):
            @pl.when(flags[b] >= _NEW_TOKEN_ID)
            def _fix():
                base = ci * _C
                liota = lax.iota(jnp.int32, 16)
                for g in range(_G):
                    idv = ids_v[pl.ds(base + g * 16, 16)]
                    m = idv == _NEW_TOKEN_ID
                    rowv = g * 16 + liota
                    for q in range(_D // 16):
                        plsc.store_scatter(
                            rows[b],
                            [rowv, q * 16 + liota],
                            new_v[pl.ds(q * 16, 16)],
                            mask=m,
                        )

        for b in range(2):
            pass1(b, b)
            fire_gathers(b)

        def pair(p, carry):
            for b in range(2):
                ci = 2 * p + b
                nci = ci + 2
                drain_gathers(b)
                fixup(ci, b)
                out_cp(ci, b).start()
                @pl.when(nci < n_chunks)
                def _prep():
                    pass1(nci, b)
                out_cp(ci, b).wait()
                @pl.when(nci < n_chunks)
                def _fire():
                    fire_gathers(b)
            return carry

        lax.fori_loop(0, n_pairs, pair, 0)

    return k(ids, table, new_row)


def kernel(input_ids, base_weight, new_weight):
    b, s = input_ids.shape
    ids = input_ids.reshape(b * s).astype(jnp.int32)
    out = _lookup(ids, base_weight, new_weight.reshape(_D), b * s)
    return out.reshape(b, s, _D)


# speedup vs baseline: 1.0477x; 1.0044x over previous
import functools

import jax
import jax.numpy as jnp
from jax import lax
from jax.experimental import pallas as pl
from jax.experimental.pallas import tpu as pltpu
from jax.experimental.pallas import tpu_sc as plsc

_NEW_TOKEN_ID = 1000000
_VOCAB = 1000000
_D = 64

_NC = 2
_NS = 16
_NW = _NC * _NS

_C = 512
_SLAB = 128
_NSLAB = _C // _SLAB
_G = _C // 16


@functools.partial(jax.jit, static_argnums=(3,))
def _lookup(ids, table, new_row, batch):
    per_w = batch // _NW
    n_chunks = per_w // _C
    n_pairs = n_chunks // 2
    mesh = plsc.VectorSubcoreMesh(core_axis_name="c", subcore_axis_name="s")

    @functools.partial(
        pl.kernel,
        mesh=mesh,
        out_type=jax.ShapeDtypeStruct((batch, _D), jnp.float32),
        scratch_types=[
            pltpu.VMEM((per_w,), jnp.int32),
            pltpu.VMEM((_C, _D), jnp.float32),
            pltpu.VMEM((_C, _D), jnp.float32),
            pltpu.VMEM((_D,), jnp.float32),
            pltpu.SMEM((2,), jnp.int32),
            pltpu.SemaphoreType.DMA,
            pltpu.SemaphoreType.DMA,
            pltpu.SemaphoreType.DMA,
            pltpu.SemaphoreType.DMA,
        ],
        compiler_params=pltpu.CompilerParams(
            needs_layout_passes=False, use_tc_tiling_on_sc=False
        ),
    )
    def k(ids_hbm, table_hbm, new_hbm, out_hbm, ids_v,
          rows0, rows1, new_v, flags, gsem0, gsem1, osem0, osem1):
        wid = lax.axis_index("s") * _NC + lax.axis_index("c")
        wbase = wid * per_w
        rows = (rows0, rows1)
        gsem = (gsem0, gsem1)
        osem = (osem0, osem1)

        pltpu.sync_copy(new_hbm, new_v)
        pltpu.sync_copy(ids_hbm.at[pl.ds(wbase, per_w)], ids_v)

        def pass1(ci, b):
            base = ci * _C
            mx = None
            for g in range(_G):
                idv = ids_v[pl.ds(base + g * 16, 16)]
                pltpu.async_copy(
                    table_hbm.at[jnp.minimum(idv, _VOCAB - 1)],
                    rows[b].at[pl.ds(g * 16, 16)],
                    gsem[b],
                )
                mx = idv if mx is None else jnp.maximum(mx, idv)
            flags[b] = jnp.max(mx)

        def drain_gathers(b):
            pltpu.make_async_copy(
                table_hbm.at[pl.ds(0, _C)], rows[b], gsem[b]
            ).wait()

        def out_cp(ci, b):
            return pltpu.make_async_copy(
                rows[b], out_hbm.at[pl.ds(wbase + ci * _C, _C)], osem[b]
            )

        def fixup(ci, b):
            @pl.when(flags[b] >= _NEW_TOKEN_ID)
            def _fix():
                base = ci * _C
                liota = lax.iota(jnp.int32, 16)
                for g in range(_G):
                    idv = ids_v[pl.ds(base + g * 16, 16)]
                    m = idv == _NEW_TOKEN_ID
                    rowv = g * 16 + liota
                    for q in range(_D // 16):
                        plsc.store_scatter(
                            rows[b],
                            [rowv, q * 16 + liota],
                            new_v[pl.ds(q * 16, 16)],
                            mask=m,
                        )

        for b in range(2):
            pass1(b, b)

        def pair(p, carry):
            for b in range(2):
                ci = 2 * p + b
                nci = ci + 2
                drain_gathers(b)
                fixup(ci, b)
                cp = out_cp(ci, b)
                cp.start()
                cp.wait()
                @pl.when(nci < n_chunks)
                def _prep():
                    pass1(nci, b)
            return carry

        lax.fori_loop(0, n_pairs, pair, 0)

    return k(ids, table, new_row)


def kernel(input_ids, base_weight, new_weight):
    b, s = input_ids.shape
    ids = input_ids.reshape(b * s).astype(jnp.int32)
    out = _lookup(ids, base_weight, new_weight.reshape(_D), b * s)
    return out.reshape(b, s, _D)
